# baseline (speedup 1.0000x reference)
import functools

import jax
import jax.numpy as jnp
from jax import lax
from jax.experimental import pallas as pl
from jax.experimental.pallas import tpu as pltpu
from jax.experimental.pallas import tpu_sc as plsc

B = 128
HW = 49
C = 512
THRESHOLD = 0.5
L = 16

_INFO = plsc.get_sparse_core_info()
NC = _INFO.num_cores
NS = _INFO.num_subcores
NW_ACTIVE = 16
SPW = B // NW_ACTIVE

_GDN = lax.GatherDimensionNumbers(
    offset_dims=(), collapsed_slice_dims=(0,), start_index_map=(0,)
)


def _lane_perm(vec, idx):
    return lax.gather(
        vec, idx[:, None], _GDN, (1,),
        mode=lax.GatherScatterMode.PROMISE_IN_BOUNDS,
    )


def _sc_body(bA_hbm, idx_hbm, bA_v, idx_v):
    wid = lax.axis_index("s") * NC + lax.axis_index("c")

    @pl.when(wid < NW_ACTIVE)
    def _():
        base = wid * SPW
        pltpu.sync_copy(bA_hbm.at[pl.ds(base, SPW)], bA_v)
        lanes = lax.iota(jnp.int32, L)
        acc = lanes
        for s in range(SPW):
            def amax_body(j, carry, s=s):
                bv, bi = carry
                v = bA_v[s, pl.ds(j * L, L)]
                idx = j * L + lanes
                take = v > bv
                return (jnp.where(take, v, bv), jnp.where(take, idx, bi))

            bv0 = bA_v[s, pl.ds(0, L)]
            bv, bi = lax.fori_loop(1, C // L, amax_body, (bv0, lanes))
            for sh in (8, 4, 2, 1):
                perm = lanes ^ sh
                pv = _lane_perm(bv, perm)
                pi = _lane_perm(bi, perm)
                take = (pv > bv) | ((pv == bv) & (pi < bi))
                bv = jnp.where(take, pv, bv)
                bi = jnp.where(take, pi, bi)
            acc = jnp.where(lanes == s, bi, acc)
        idx_v[...] = acc
        pltpu.sync_copy(idx_v.at[pl.ds(0, SPW)], idx_hbm.at[pl.ds(base, SPW)])


_sc_argmax = functools.partial(
    pl.kernel,
    out_type=jax.ShapeDtypeStruct((B,), jnp.int32),
    mesh=plsc.VectorSubcoreMesh(core_axis_name="c", subcore_axis_name="s"),
    scratch_types=[
        pltpu.VMEM((SPW, C), jnp.float32),
        pltpu.VMEM((L,), jnp.int32),
    ],
)(_sc_body)


def _tc_body(idx_ref, interm_ref, vgg_ref, o_ref):
    i = pl.program_id(0)
    sel = idx_ref[i] % 128
    hot = lax.broadcasted_iota(jnp.int32, (1, 7, 7, 128), 3) == sel
    a = jnp.sum(jnp.where(hot, interm_ref[...], 0.0), axis=3, keepdims=True)
    tmp = jnp.where(a > THRESHOLD, a, 0.0)
    o_ref[...] = vgg_ref[...] - tmp


def kernel(vgg_end, interm, branchA_end):
    idx = _sc_argmax(branchA_end)
    grid_spec = pltpu.PrefetchScalarGridSpec(
        num_scalar_prefetch=1,
        grid=(B,),
        in_specs=[
            pl.BlockSpec((1, 7, 7, 128), lambda i, idx_ref: (i, 0, 0, idx_ref[i] // 128)),
            pl.BlockSpec((1, 7, 7, C), lambda i, idx_ref: (i, 0, 0, 0)),
        ],
        out_specs=pl.BlockSpec((1, 7, 7, C), lambda i, idx_ref: (i, 0, 0, 0)),
    )
    return pl.pallas_call(
        _tc_body,
        grid_spec=grid_spec,
        out_shape=jax.ShapeDtypeStruct((B, 7, 7, C), jnp.float32),
    )(idx, interm, vgg_end)


# speedup vs baseline: 1.6331x; 1.6331x over previous
import functools

import jax
import jax.numpy as jnp
from jax import lax
from jax.experimental import pallas as pl
from jax.experimental.pallas import tpu as pltpu
from jax.experimental.pallas import tpu_sc as plsc

B = 128
HW = 49
C = 512
THRESHOLD = 0.5
L = 16

_INFO = plsc.get_sparse_core_info()
NC = _INFO.num_cores
NS = _INFO.num_subcores
NW_ACTIVE = 16
SPW = B // NW_ACTIVE

_GDN = lax.GatherDimensionNumbers(
    offset_dims=(), collapsed_slice_dims=(0,), start_index_map=(0,)
)


def _lane_perm(vec, idx):
    return lax.gather(
        vec, idx[:, None], _GDN, (1,),
        mode=lax.GatherScatterMode.PROMISE_IN_BOUNDS,
    )


def _sc_body(bA_hbm, idx_hbm, bA_v, idx_v):
    wid = lax.axis_index("s") * NC + lax.axis_index("c")

    @pl.when(wid < NW_ACTIVE)
    def _():
        base = wid * SPW
        pltpu.sync_copy(bA_hbm.at[pl.ds(base, SPW)], bA_v)
        lanes = lax.iota(jnp.int32, L)
        acc = lanes
        for s in range(SPW):
            def amax_body(j, carry, s=s):
                bv, bi = carry
                v = bA_v[s, pl.ds(j * L, L)]
                idx = j * L + lanes
                take = v > bv
                return (jnp.where(take, v, bv), jnp.where(take, idx, bi))

            bv0 = bA_v[s, pl.ds(0, L)]
            bv, bi = lax.fori_loop(1, C // L, amax_body, (bv0, lanes))
            for sh in (8, 4, 2, 1):
                perm = lanes ^ sh
                pv = _lane_perm(bv, perm)
                pi = _lane_perm(bi, perm)
                take = (pv > bv) | ((pv == bv) & (pi < bi))
                bv = jnp.where(take, pv, bv)
                bi = jnp.where(take, pi, bi)
            acc = jnp.where(lanes == s, bi, acc)
        idx_v[...] = acc
        pltpu.sync_copy(idx_v.at[pl.ds(0, SPW)], idx_hbm.at[pl.ds(base, SPW)])


_sc_argmax = functools.partial(
    pl.kernel,
    out_type=jax.ShapeDtypeStruct((B,), jnp.int32),
    mesh=plsc.VectorSubcoreMesh(core_axis_name="c", subcore_axis_name="s"),
    scratch_types=[
        pltpu.VMEM((SPW, C), jnp.float32),
        pltpu.VMEM((L,), jnp.int32),
    ],
)(_sc_body)


SPB = 8
IOTA_C = None


def _tc_body(idx_ref, interm_ref, vgg_ref, o_ref):
    i = pl.program_id(0)
    iota_c = lax.broadcasted_iota(jnp.int32, (7, 7, C), 2)
    for s in range(SPB):
        sel = idx_ref[i * SPB + s]
        a = jnp.sum(
            jnp.where(iota_c == sel, interm_ref[s], 0.0),
            axis=2, keepdims=True,
        )
        tmp = jnp.where(a > THRESHOLD, a, 0.0)
        o_ref[s] = vgg_ref[s] - tmp


def kernel(vgg_end, interm, branchA_end):
    idx = _sc_argmax(branchA_end)
    grid_spec = pltpu.PrefetchScalarGridSpec(
        num_scalar_prefetch=1,
        grid=(B // SPB,),
        in_specs=[
            pl.BlockSpec((SPB, 7, 7, C), lambda i, idx_ref: (i, 0, 0, 0)),
            pl.BlockSpec((SPB, 7, 7, C), lambda i, idx_ref: (i, 0, 0, 0)),
        ],
        out_specs=pl.BlockSpec((SPB, 7, 7, C), lambda i, idx_ref: (i, 0, 0, 0)),
    )
    return pl.pallas_call(
        _tc_body,
        grid_spec=grid_spec,
        out_shape=jax.ShapeDtypeStruct((B, 7, 7, C), jnp.float32),
    )(idx, interm, vgg_end)
